# Initial kernel scaffold; baseline (speedup 1.0000x reference)
import functools

import jax
import jax.numpy as jnp
from jax import lax
from jax.experimental import pallas as pl
from jax.experimental.pallas import tpu as pltpu
from jax.experimental.pallas import tpu_sc as plsc

_EPS = 1e-5
_NC = 2
_NS = 16


@functools.lru_cache(maxsize=None)
def _make_sc_gather(bf, d, chunk, n_chunks, per_w):
    mesh = plsc.VectorSubcoreMesh(core_axis_name="c", subcore_axis_name="s")

    @functools.partial(
        pl.kernel,
        mesh=mesh,
        out_type=jax.ShapeDtypeStruct((bf, d), jnp.float32),
        scratch_types=[
            pltpu.VMEM((chunk,), jnp.int32),
            pltpu.VMEM((chunk, d), jnp.float32),
            pltpu.SemaphoreType.DMA,
        ],
    )
    def gather_k(table_hbm, idx_hbm, out_hbm, idx_v, rows_v, sem):
        wid = lax.axis_index("s") * _NC + lax.axis_index("c")
        base = wid * per_w

        def body(i, carry):
            off = base + i * chunk
            pltpu.sync_copy(idx_hbm.at[pl.ds(off, chunk)], idx_v)
            pltpu.async_copy(table_hbm.at[idx_v], rows_v, sem).wait()
            pltpu.sync_copy(rows_v, out_hbm.at[pl.ds(off, chunk)])
            return carry

        lax.fori_loop(0, n_chunks, body, 0)

    return gather_k


def _mlp_body(xn_ref, emb_ref, w1n_ref, w1e_ref, w2_ref, w3_ref, w4_ref,
              b1_ref, g1_ref, be1_ref, b2_ref, g2_ref, be2_ref,
              b3_ref, g3_ref, be3_ref, b4_ref, out_ref):
    rinv = (1.0 + _EPS) ** -0.5
    y1 = jnp.dot(xn_ref[...], w1n_ref[...], preferred_element_type=jnp.float32)
    y1 += jnp.dot(emb_ref[...], w1e_ref[...], preferred_element_type=jnp.float32)
    h1 = jnp.maximum((y1 + b1_ref[...]) * (g1_ref[...] * rinv) + be1_ref[...], 0.0)
    y2 = jnp.dot(h1, w2_ref[...], preferred_element_type=jnp.float32)
    h2 = jnp.maximum((y2 + b2_ref[...]) * (g2_ref[...] * rinv) + be2_ref[...], 0.0)
    y3 = jnp.dot(h2, w3_ref[...], preferred_element_type=jnp.float32)
    h3 = jnp.maximum((y3 + b3_ref[...]) * (g3_ref[...] * rinv) + be3_ref[...], 0.0)
    y4 = jnp.dot(h3, w4_ref[...], preferred_element_type=jnp.float32) + b4_ref[...]
    out_ref[...] = jax.nn.sigmoid(y4)


def kernel(x_num, x_cat, tables, W1, b1, g1, be1, W2, b2, g2, be2,
           W3, b3, g3, be3, W4, b4):
    B, num = x_num.shape
    F = x_cat.shape[1]
    V, D = tables.shape[1], tables.shape[2]
    bf = B * F

    nw = _NC * _NS
    per_w = bf // nw
    chunk = 1664
    n_chunks = per_w // chunk
    flat_tab = tables.reshape(F * V, D)
    idx = (x_cat.astype(jnp.int32)
           + (jnp.arange(F, dtype=jnp.int32) * V)[None, :]).reshape(bf)
    emb_flat = _make_sc_gather(bf, D, chunk, n_chunks, per_w)(flat_tab, idx)
    emb = emb_flat.reshape(B, F * D)

    bm = 2048
    d1, d2, d3 = W1.shape[0], W2.shape[0], W3.shape[0]
    row = lambda r, c: pl.BlockSpec((r, c), lambda i: (0, 0))
    out = pl.pallas_call(
        _mlp_body,
        grid=(B // bm,),
        in_specs=[
            pl.BlockSpec((bm, num), lambda i: (i, 0)),
            pl.BlockSpec((bm, F * D), lambda i: (i, 0)),
            row(num, d1), row(F * D, d1), row(d1, d2), row(d2, d3), row(d3, 1),
            row(1, d1), row(1, d1), row(1, d1),
            row(1, d2), row(1, d2), row(1, d2),
            row(1, d3), row(1, d3), row(1, d3),
            row(1, 1),
        ],
        out_specs=pl.BlockSpec((bm, 1), lambda i: (i, 0)),
        out_shape=jax.ShapeDtypeStruct((B, 1), jnp.float32),
    )(
        x_num, emb,
        W1[:, :num].T, W1[:, num:].T, W2.T, W3.T, W4.T,
        b1[None, :], g1[None, :], be1[None, :],
        b2[None, :], g2[None, :], be2[None, :],
        b3[None, :], g3[None, :], be3[None, :],
        b4[None, :],
    )
    return out


# speedup vs baseline: 7.7174x; 7.7174x over previous
import functools

import jax
import jax.numpy as jnp
from jax import lax
from jax.experimental import pallas as pl
from jax.experimental.pallas import tpu as pltpu
from jax.experimental.pallas import tpu_sc as plsc

_EPS = 1e-5
_NC = 2
_NS = 16


@functools.lru_cache(maxsize=None)
def _make_sc_gather(bf, d, chunk, n_chunks, per_w):
    mesh = plsc.VectorSubcoreMesh(core_axis_name="c", subcore_axis_name="s")

    @functools.partial(
        pl.kernel,
        mesh=mesh,
        out_type=jax.ShapeDtypeStruct((bf, d), jnp.float32),
        compiler_params=pltpu.CompilerParams(use_tc_tiling_on_sc=False),
        scratch_types=[
            pltpu.VMEM((chunk,), jnp.int32),
            pltpu.VMEM((chunk, d), jnp.float32),
            pltpu.SemaphoreType.DMA,
        ],
    )
    def gather_k(table_hbm, idx_hbm, out_hbm, idx_v, rows_v, sem):
        wid = lax.axis_index("s") * _NC + lax.axis_index("c")
        base = wid * per_w

        def body(i, carry):
            off = base + i * chunk
            pltpu.sync_copy(idx_hbm.at[pl.ds(off, chunk)], idx_v)
            pltpu.async_copy(table_hbm.at[idx_v], rows_v, sem).wait()
            pltpu.sync_copy(rows_v, out_hbm.at[pl.ds(off, chunk)])
            return carry

        lax.fori_loop(0, n_chunks, body, 0)

    return gather_k


def _mlp_body(xn_ref, emb_ref, w1n_ref, w1e_ref, w2_ref, w3_ref, w4_ref,
              b1_ref, g1_ref, be1_ref, b2_ref, g2_ref, be2_ref,
              b3_ref, g3_ref, be3_ref, b4_ref, out_ref):
    rinv = (1.0 + _EPS) ** -0.5
    y1 = jnp.dot(xn_ref[...], w1n_ref[...], preferred_element_type=jnp.float32)
    y1 += jnp.dot(emb_ref[...], w1e_ref[...], preferred_element_type=jnp.float32)
    h1 = jnp.maximum((y1 + b1_ref[...]) * (g1_ref[...] * rinv) + be1_ref[...], 0.0)
    y2 = jnp.dot(h1, w2_ref[...], preferred_element_type=jnp.float32)
    h2 = jnp.maximum((y2 + b2_ref[...]) * (g2_ref[...] * rinv) + be2_ref[...], 0.0)
    y3 = jnp.dot(h2, w3_ref[...], preferred_element_type=jnp.float32)
    h3 = jnp.maximum((y3 + b3_ref[...]) * (g3_ref[...] * rinv) + be3_ref[...], 0.0)
    y4 = jnp.dot(h3, w4_ref[...], preferred_element_type=jnp.float32) + b4_ref[...]
    out_ref[...] = jax.nn.sigmoid(y4)


def kernel(x_num, x_cat, tables, W1, b1, g1, be1, W2, b2, g2, be2,
           W3, b3, g3, be3, W4, b4):
    B, num = x_num.shape
    F = x_cat.shape[1]
    V, D = tables.shape[1], tables.shape[2]
    bf = B * F

    nw = _NC * _NS
    per_w = bf // nw
    chunk = 1664
    n_chunks = per_w // chunk
    flat_tab = tables.reshape(F * V, D)
    idx = (x_cat.astype(jnp.int32)
           + (jnp.arange(F, dtype=jnp.int32) * V)[None, :]).reshape(bf)
    emb_flat = _make_sc_gather(bf, D, chunk, n_chunks, per_w)(flat_tab, idx)
    emb = emb_flat.reshape(B, F * D)

    bm = 2048
    d1, d2, d3 = W1.shape[0], W2.shape[0], W3.shape[0]
    row = lambda r, c: pl.BlockSpec((r, c), lambda i: (0, 0))
    out = pl.pallas_call(
        _mlp_body,
        grid=(B // bm,),
        in_specs=[
            pl.BlockSpec((bm, num), lambda i: (i, 0)),
            pl.BlockSpec((bm, F * D), lambda i: (i, 0)),
            row(num, d1), row(F * D, d1), row(d1, d2), row(d2, d3), row(d3, 1),
            row(1, d1), row(1, d1), row(1, d1),
            row(1, d2), row(1, d2), row(1, d2),
            row(1, d3), row(1, d3), row(1, d3),
            row(1, 1),
        ],
        out_specs=pl.BlockSpec((bm, 1), lambda i: (i, 0)),
        out_shape=jax.ShapeDtypeStruct((B, 1), jnp.float32),
    )(
        x_num, emb,
        W1[:, :num].T, W1[:, num:].T, W2.T, W3.T, W4.T,
        b1[None, :], g1[None, :], be1[None, :],
        b2[None, :], g2[None, :], be2[None, :],
        b3[None, :], g3[None, :], be3[None, :],
        b4[None, :],
    )
    return out


# speedup vs baseline: 45.7032x; 5.9221x over previous
import functools

import jax
import jax.numpy as jnp
from jax import lax
from jax.experimental import pallas as pl
from jax.experimental.pallas import tpu as pltpu
from jax.experimental.pallas import tpu_sc as plsc

_EPS = 1e-5
_NC = 2
_NS = 16


@functools.lru_cache(maxsize=None)
def _make_sc_gather_t(F, V, D, B, units_per_w, chunk):
    mesh = plsc.VectorSubcoreMesh(core_axis_name="c", subcore_axis_name="s")
    n_chunks = B // chunk

    @functools.partial(
        pl.kernel,
        mesh=mesh,
        out_type=jax.ShapeDtypeStruct((F * D, B), jnp.float32),
        scratch_types=(
            [pltpu.VMEM_SHARED((V,), jnp.float32) for _ in range(_NS)]
            + [pltpu.VMEM((chunk,), jnp.int32),
               pltpu.VMEM((chunk,), jnp.float32),
               pltpu.SemaphoreType.DMA]
        ),
    )
    def gather_k(tab_hbm, idx_hbm, out_hbm, *refs):
        planes = refs[:_NS]
        idx_v, val_v, sem = refs[_NS], refs[_NS + 1], refs[_NS + 2]
        cid = lax.axis_index("c")
        sid = lax.axis_index("s")
        wid = sid * _NC + cid

        def work(plane):
            def unit_body(k, carry):
                u = wid * units_per_w + k
                f = u // D
                d = u % D
                pltpu.sync_copy(tab_hbm.at[f, d], plane)

                def chunk_body(c, carry2):
                    b0 = c * chunk
                    pltpu.sync_copy(idx_hbm.at[f, pl.ds(b0, chunk)], idx_v)
                    pltpu.async_copy(plane.at[idx_v], val_v, sem).wait()
                    pltpu.sync_copy(val_v, out_hbm.at[u, pl.ds(b0, chunk)])
                    return carry2

                lax.fori_loop(0, n_chunks, chunk_body, 0)
                return carry

            lax.fori_loop(0, units_per_w, unit_body, 0)

        for t in range(_NS):
            pl.when(sid == t)(functools.partial(work, planes[t]))

    return gather_k


def _mlp_t_body(xnt_ref, embt_ref, w1n_ref, w1e_ref, w2_ref, w3_ref, w4_ref,
                b1_ref, g1_ref, be1_ref, b2_ref, g2_ref, be2_ref,
                b3_ref, g3_ref, be3_ref, b4_ref, out_ref):
    rinv = (1.0 + _EPS) ** -0.5
    y1 = jnp.dot(w1n_ref[...], xnt_ref[...], preferred_element_type=jnp.float32)
    y1 += jnp.dot(w1e_ref[...], embt_ref[...], preferred_element_type=jnp.float32)
    h1 = jnp.maximum((y1 + b1_ref[...]) * (g1_ref[...] * rinv) + be1_ref[...], 0.0)
    y2 = jnp.dot(w2_ref[...], h1, preferred_element_type=jnp.float32)
    h2 = jnp.maximum((y2 + b2_ref[...]) * (g2_ref[...] * rinv) + be2_ref[...], 0.0)
    y3 = jnp.dot(w3_ref[...], h2, preferred_element_type=jnp.float32)
    h3 = jnp.maximum((y3 + b3_ref[...]) * (g3_ref[...] * rinv) + be3_ref[...], 0.0)
    y4 = jnp.dot(w4_ref[...], h3, preferred_element_type=jnp.float32) + b4_ref[...]
    out_ref[...] = jax.nn.sigmoid(y4)


def kernel(x_num, x_cat, tables, W1, b1, g1, be1, W2, b2, g2, be2,
           W3, b3, g3, be3, W4, b4):
    B, num = x_num.shape
    F = x_cat.shape[1]
    V, D = tables.shape[1], tables.shape[2]

    nw = _NC * _NS
    units_per_w = (F * D) // nw
    chunk = 4096
    tab_t = jnp.transpose(tables, (0, 2, 1))
    idx_t = jnp.transpose(x_cat.astype(jnp.int32), (1, 0))
    emb_t = _make_sc_gather_t(F, V, D, B, units_per_w, chunk)(tab_t, idx_t)

    bm = 2048
    d1, d2, d3 = W1.shape[0], W2.shape[0], W3.shape[0]
    cst = lambda r, c: pl.BlockSpec((r, c), lambda i: (0, 0))
    out_t = pl.pallas_call(
        _mlp_t_body,
        grid=(B // bm,),
        in_specs=[
            pl.BlockSpec((num, bm), lambda i: (0, i)),
            pl.BlockSpec((F * D, bm), lambda i: (0, i)),
            cst(d1, num), cst(d1, F * D), cst(d2, d1), cst(d3, d2), cst(1, d3),
            cst(d1, 1), cst(d1, 1), cst(d1, 1),
            cst(d2, 1), cst(d2, 1), cst(d2, 1),
            cst(d3, 1), cst(d3, 1), cst(d3, 1),
            cst(1, 1),
        ],
        out_specs=pl.BlockSpec((1, bm), lambda i: (0, i)),
        out_shape=jax.ShapeDtypeStruct((1, B), jnp.float32),
    )(
        x_num.T, emb_t,
        W1[:, :num], W1[:, num:], W2, W3, W4,
        b1[:, None], g1[:, None], be1[:, None],
        b2[:, None], g2[:, None], be2[:, None],
        b3[:, None], g3[:, None], be3[:, None],
        b4[:, None],
    )
    return out_t.reshape(B, 1)


# speedup vs baseline: 55.6298x; 1.2172x over previous
import functools

import jax
import jax.numpy as jnp
from jax import lax
from jax.experimental import pallas as pl
from jax.experimental.pallas import tpu as pltpu
from jax.experimental.pallas import tpu_sc as plsc

_EPS = 1e-5
_NC = 2
_NS = 16


@functools.lru_cache(maxsize=None)
def _make_sc_gather_t(F, V, D, B, units_per_w, chunk):
    mesh = plsc.VectorSubcoreMesh(core_axis_name="c", subcore_axis_name="s")
    U = units_per_w
    NCH = B // chunk
    steps = [(k, c) for k in range(U) for c in range(NCH)]

    @functools.partial(
        pl.kernel,
        mesh=mesh,
        out_type=jax.ShapeDtypeStruct((F * D, B), jnp.float32),
        scratch_types=(
            [pltpu.VMEM_SHARED((V,), jnp.float32) for _ in range(_NS)]
            + [pltpu.VMEM((chunk,), jnp.int32)]
            + [pltpu.VMEM((chunk,), jnp.float32) for _ in range(2)]
            + [pltpu.SemaphoreType.DMA for _ in range(4)]
        ),
    )
    def gather_k(tab_hbm, idx_hbm, out_hbm, *refs):
        planes = refs[:_NS]
        idx_b = refs[_NS]
        val_b = refs[_NS + 1:_NS + 3]
        psem = refs[_NS + 3]
        gsem = refs[_NS + 4]
        osems = refs[_NS + 5:_NS + 7]
        cid = lax.axis_index("c")
        sid = lax.axis_index("s")
        wid = sid * _NC + cid

        def fd(k):
            u = wid * U + k
            return u, u // D, u % D

        _, f0, d0 = fd(0)
        for t in range(_NS):
            def _load0(t=t):
                pltpu.async_copy(tab_hbm.at[f0, d0], planes[t], psem)
            pl.when(sid == t)(_load0)
        out_h = {}

        for s, (k, c) in enumerate(steps):
            u, f, d = fd(k)
            if s >= 2:
                out_h[s - 2].wait()
            pltpu.sync_copy(idx_hbm.at[f, pl.ds(c * chunk, chunk)], idx_b)
            for t in range(_NS):
                def _gather(t=t, k=k, c=c, s=s, f=f, d=d):
                    if c == 0:
                        pltpu.make_async_copy(
                            tab_hbm.at[f, d], planes[t], psem).wait()
                    pltpu.async_copy(
                        planes[t].at[idx_b], val_b[s % 2], gsem).wait()
                    if c == NCH - 1 and k + 1 < U:
                        _, f1, d1 = fd(k + 1)
                        pltpu.async_copy(tab_hbm.at[f1, d1], planes[t], psem)
                pl.when(sid == t)(_gather)
            out_h[s] = pltpu.async_copy(
                val_b[s % 2], out_hbm.at[u, pl.ds(c * chunk, chunk)],
                osems[s % 2])

        out_h[len(steps) - 2].wait()
        out_h[len(steps) - 1].wait()

    return gather_k


def _mlp_t_body(xnt_ref, embt_ref, w1n_ref, w1e_ref, w2_ref, w3_ref, w4_ref,
                b1_ref, g1_ref, be1_ref, b2_ref, g2_ref, be2_ref,
                b3_ref, g3_ref, be3_ref, b4_ref, out_ref):
    rinv = (1.0 + _EPS) ** -0.5
    y1 = jnp.dot(w1n_ref[...], xnt_ref[...], preferred_element_type=jnp.float32)
    y1 += jnp.dot(w1e_ref[...], embt_ref[...], preferred_element_type=jnp.float32)
    h1 = jnp.maximum((y1 + b1_ref[...]) * (g1_ref[...] * rinv) + be1_ref[...], 0.0)
    y2 = jnp.dot(w2_ref[...], h1, preferred_element_type=jnp.float32)
    h2 = jnp.maximum((y2 + b2_ref[...]) * (g2_ref[...] * rinv) + be2_ref[...], 0.0)
    y3 = jnp.dot(w3_ref[...], h2, preferred_element_type=jnp.float32)
    h3 = jnp.maximum((y3 + b3_ref[...]) * (g3_ref[...] * rinv) + be3_ref[...], 0.0)
    y4 = jnp.dot(w4_ref[...], h3, preferred_element_type=jnp.float32) + b4_ref[...]
    out_ref[...] = jax.nn.sigmoid(y4)


def kernel(x_num, x_cat, tables, W1, b1, g1, be1, W2, b2, g2, be2,
           W3, b3, g3, be3, W4, b4):
    B, num = x_num.shape
    F = x_cat.shape[1]
    V, D = tables.shape[1], tables.shape[2]

    nw = _NC * _NS
    units_per_w = (F * D) // nw
    tab_t = jnp.transpose(tables, (0, 2, 1))
    idx_t = jnp.transpose(x_cat.astype(jnp.int32), (1, 0))
    emb_t = _make_sc_gather_t(F, V, D, B, units_per_w, 8192)(tab_t, idx_t)

    bm = 2048
    d1, d2, d3 = W1.shape[0], W2.shape[0], W3.shape[0]
    cst = lambda r, c: pl.BlockSpec((r, c), lambda i: (0, 0))
    out_t = pl.pallas_call(
        _mlp_t_body,
        grid=(B // bm,),
        in_specs=[
            pl.BlockSpec((num, bm), lambda i: (0, i)),
            pl.BlockSpec((F * D, bm), lambda i: (0, i)),
            cst(d1, num), cst(d1, F * D), cst(d2, d1), cst(d3, d2), cst(1, d3),
            cst(d1, 1), cst(d1, 1), cst(d1, 1),
            cst(d2, 1), cst(d2, 1), cst(d2, 1),
            cst(d3, 1), cst(d3, 1), cst(d3, 1),
            cst(1, 1),
        ],
        out_specs=pl.BlockSpec((1, bm), lambda i: (0, i)),
        out_shape=jax.ShapeDtypeStruct((1, B), jnp.float32),
    )(
        x_num.T, emb_t,
        W1[:, :num], W1[:, num:], W2, W3, W4,
        b1[:, None], g1[:, None], be1[:, None],
        b2[:, None], g2[:, None], be2[:, None],
        b3[:, None], g3[:, None], be3[:, None],
        b4[:, None],
    )
    return out_t.reshape(B, 1)


# speedup vs baseline: 56.6247x; 1.0179x over previous
import functools

import jax
import jax.numpy as jnp
from jax import lax
from jax.experimental import pallas as pl
from jax.experimental.pallas import tpu as pltpu
from jax.experimental.pallas import tpu_sc as plsc

_EPS = 1e-5
_NC = 2
_NS = 16


@functools.lru_cache(maxsize=None)
def _make_sc_gather_t(F, V, D, B, units_per_w, chunk):
    mesh = plsc.VectorSubcoreMesh(core_axis_name="c", subcore_axis_name="s")
    U = units_per_w
    NCH = B // chunk
    steps = [(k, c) for k in range(U) for c in range(NCH)]

    @functools.partial(
        pl.kernel,
        mesh=mesh,
        out_type=jax.ShapeDtypeStruct((F * D, B), jnp.float32),
        scratch_types=(
            [pltpu.VMEM_SHARED((V,), jnp.float32) for _ in range(_NS)]
            + [pltpu.VMEM((chunk,), jnp.int32)]
            + [pltpu.VMEM((chunk,), jnp.float32) for _ in range(2)]
            + [pltpu.SemaphoreType.DMA for _ in range(4)]
        ),
    )
    def gather_k(tab_hbm, idx_hbm, out_hbm, *refs):
        planes = refs[:_NS]
        idx_b = refs[_NS]
        val_b = refs[_NS + 1:_NS + 3]
        psem = refs[_NS + 3]
        gsem = refs[_NS + 4]
        osems = refs[_NS + 5:_NS + 7]
        cid = lax.axis_index("c")
        sid = lax.axis_index("s")
        wid = sid * _NC + cid

        def fd(k):
            u = wid * U + k
            return u, u // D, u % D

        _, f0, d0 = fd(0)
        for t in range(_NS):
            def _load0(t=t):
                pltpu.async_copy(tab_hbm.at[f0, d0], planes[t], psem)
            pl.when(sid == t)(_load0)
        out_h = {}

        for s, (k, c) in enumerate(steps):
            u, f, d = fd(k)
            if s >= 2:
                out_h[s - 2].wait()
            pltpu.sync_copy(idx_hbm.at[f, pl.ds(c * chunk, chunk)], idx_b)
            for t in range(_NS):
                def _gather(t=t, k=k, c=c, s=s, f=f, d=d):
                    if c == 0:
                        pltpu.make_async_copy(
                            tab_hbm.at[f, d], planes[t], psem).wait()
                    pltpu.async_copy(
                        planes[t].at[idx_b], val_b[s % 2], gsem).wait()
                    if c == NCH - 1 and k + 1 < U:
                        _, f1, d1 = fd(k + 1)
                        pltpu.async_copy(tab_hbm.at[f1, d1], planes[t], psem)
                pl.when(sid == t)(_gather)
            out_h[s] = pltpu.async_copy(
                val_b[s % 2], out_hbm.at[u, pl.ds(c * chunk, chunk)],
                osems[s % 2])

        out_h[len(steps) - 2].wait()
        out_h[len(steps) - 1].wait()

    return gather_k


def _mlp_t_body(xnt_ref, embt_ref, w1n_ref, w1e_ref, w2_ref, w3_ref, w4_ref,
                b1_ref, g1_ref, be1_ref, b2_ref, g2_ref, be2_ref,
                b3_ref, g3_ref, be3_ref, b4_ref, out_ref):
    rinv = (1.0 + _EPS) ** -0.5
    y1 = jnp.dot(w1n_ref[...], xnt_ref[...], preferred_element_type=jnp.float32)
    y1 += jnp.dot(w1e_ref[...], embt_ref[...], preferred_element_type=jnp.float32)
    h1 = jnp.maximum((y1 + b1_ref[...]) * (g1_ref[...] * rinv) + be1_ref[...], 0.0)
    y2 = jnp.dot(w2_ref[...], h1, preferred_element_type=jnp.float32)
    h2 = jnp.maximum((y2 + b2_ref[...]) * (g2_ref[...] * rinv) + be2_ref[...], 0.0)
    y3 = jnp.dot(w3_ref[...], h2, preferred_element_type=jnp.float32)
    h3 = jnp.maximum((y3 + b3_ref[...]) * (g3_ref[...] * rinv) + be3_ref[...], 0.0)
    y4 = jnp.dot(w4_ref[...], h3, preferred_element_type=jnp.float32) + b4_ref[...]
    out_ref[...] = jax.nn.sigmoid(y4)


def kernel(x_num, x_cat, tables, W1, b1, g1, be1, W2, b2, g2, be2,
           W3, b3, g3, be3, W4, b4):
    B, num = x_num.shape
    F = x_cat.shape[1]
    V, D = tables.shape[1], tables.shape[2]

    nw = _NC * _NS
    units_per_w = (F * D) // nw
    tab_t = jnp.transpose(tables, (0, 2, 1))
    idx_t = jnp.transpose(x_cat.astype(jnp.int32), (1, 0))
    emb_t = _make_sc_gather_t(F, V, D, B, units_per_w, 8192)(tab_t, idx_t)

    bm = 4096
    d1, d2, d3 = W1.shape[0], W2.shape[0], W3.shape[0]
    cst = lambda r, c: pl.BlockSpec((r, c), lambda i: (0, 0))
    out_t = pl.pallas_call(
        _mlp_t_body,
        grid=(B // bm,),
        in_specs=[
            pl.BlockSpec((num, bm), lambda i: (0, i)),
            pl.BlockSpec((F * D, bm), lambda i: (0, i)),
            cst(d1, num), cst(d1, F * D), cst(d2, d1), cst(d3, d2), cst(1, d3),
            cst(d1, 1), cst(d1, 1), cst(d1, 1),
            cst(d2, 1), cst(d2, 1), cst(d2, 1),
            cst(d3, 1), cst(d3, 1), cst(d3, 1),
            cst(1, 1),
        ],
        out_specs=pl.BlockSpec((1, bm), lambda i: (0, i)),
        out_shape=jax.ShapeDtypeStruct((1, B), jnp.float32),
    )(
        x_num.T, emb_t,
        W1[:, :num], W1[:, num:], W2, W3, W4,
        b1[:, None], g1[:, None], be1[:, None],
        b2[:, None], g2[:, None], be2[:, None],
        b3[:, None], g3[:, None], be3[:, None],
        b4[:, None],
    )
    return out_t.reshape(B, 1)
